# Initial kernel scaffold; baseline (speedup 1.0000x reference)
import jax
import jax.numpy as jnp
from jax.experimental import pallas as pl


def kernel(x, edge_index, batch, W1, b1, W2, b2, W3, b3):
    raise NotImplementedError("write your pallas kernel here")


# speedup vs baseline: 7.5931x; 7.5931x over previous
import functools

import jax
import jax.numpy as jnp
from jax import lax
from jax.experimental import pallas as pl
from jax.experimental.pallas import tpu as pltpu
from jax.experimental.pallas import tpu_sc as plsc

N = 10000
E = 160000
D_IN = 256
H = 512
K = 128
NCHUNK = E // K
TN = 400
GRID = N // TN

_mesh = plsc.VectorSubcoreMesh(core_axis_name="c", subcore_axis_name="s")


def _zero_rows(z_h, acc, sid):
    @pl.when(sid < 15)
    def _():
        pltpu.sync_copy(z_h.at[pl.ds(sid * 640, 640)],
                        acc.at[pl.ds(sid * 640, 640)])

    @pl.when(sid == 15)
    def _():
        pltpu.sync_copy(z_h.at[pl.ds(9600, 400)], acc.at[pl.ds(9600, 400)])


def _sc_degree(dst, ones_nk, z_nk):
    @functools.partial(
        pl.kernel,
        out_type=jax.ShapeDtypeStruct((2, N, 128), jnp.float32),
        mesh=_mesh,
        scratch_types=[
            pltpu.VMEM((K,), jnp.int32),
            pltpu.VMEM((K, 128), jnp.float32),
            pltpu.VMEM_SHARED((N, 128), jnp.float32),
        ],
    )
    def k(dst_h, ones_h, z_h, out_h, dst_v, msgs_v, acc):
        c = lax.axis_index("c")
        sid = lax.axis_index("s")
        _zero_rows(z_h, acc, sid)
        plsc.subcore_barrier()

        base = c * (NCHUNK // 2)

        def chunk(cid):
            pltpu.sync_copy(dst_h.at[pl.ds(cid * K, K)], dst_v)
            pltpu.sync_copy(ones_h.at[dst_v], msgs_v)
            pltpu.sync_copy(msgs_v, acc.at[dst_v], add=True)

        @pl.loop(0, 39)
        def _(r):
            chunk(base + r * 16 + sid)

        @pl.when(sid == 0)
        def _():
            chunk(base + 624)

        plsc.subcore_barrier()

        @pl.when(sid < 15)
        def _():
            pltpu.sync_copy(acc.at[pl.ds(sid * 640, 640)],
                            out_h.at[c].at[pl.ds(sid * 640, 640)])

        @pl.when(sid == 15)
        def _():
            pltpu.sync_copy(acc.at[pl.ds(9600, 400)],
                            out_h.at[c].at[pl.ds(9600, 400)])

    return k(dst, ones_nk, z_nk)


def _sc_agg(table, src, dst, z_nk, C):
    @functools.partial(
        pl.kernel,
        out_type=jax.ShapeDtypeStruct((C, N, 128), jnp.float32),
        mesh=_mesh,
        scratch_types=[
            pltpu.VMEM((K,), jnp.int32),
            pltpu.VMEM((K,), jnp.int32),
            pltpu.VMEM((K, 128), jnp.float32),
            pltpu.VMEM_SHARED((N, 128), jnp.float32),
        ],
    )
    def k(table_h, src_h, dst_h, z_h, out_h, src_v, dst_v, msgs_v, acc):
        c = lax.axis_index("c")
        sid = lax.axis_index("s")
        for qq in range(C // 2):
            q = c * (C // 2) + qq
            _zero_rows(z_h, acc, sid)
            plsc.subcore_barrier()

            def chunk(cid):
                pltpu.sync_copy(src_h.at[pl.ds(cid * K, K)], src_v)
                pltpu.sync_copy(dst_h.at[pl.ds(cid * K, K)], dst_v)
                pltpu.sync_copy(table_h.at[q].at[src_v], msgs_v)
                pltpu.sync_copy(msgs_v, acc.at[dst_v], add=True)

            @pl.loop(0, NCHUNK // 16)
            def _(r):
                chunk(r * 16 + sid)

            @pl.when(sid < NCHUNK - (NCHUNK // 16) * 16)
            def _():
                chunk((NCHUNK // 16) * 16 + sid)

            plsc.subcore_barrier()

            @pl.when(sid < 15)
            def _():
                pltpu.sync_copy(acc.at[pl.ds(sid * 640, 640)],
                                out_h.at[q].at[pl.ds(sid * 640, 640)])

            @pl.when(sid == 15)
            def _():
                pltpu.sync_copy(acc.at[pl.ds(9600, 400)],
                                out_h.at[q].at[pl.ds(9600, 400)])

            if qq + 1 < C // 2:
                plsc.subcore_barrier()

    return k(table, src, dst, z_nk)


def _dis(deg_ref):
    deg = deg_ref[:, 0:1] + deg_ref[:, 1:2] + 1.0
    return lax.rsqrt(deg)


def _tc_pre(degt, x):
    def body(deg_ref, x_ref, xq_ref):
        xs = x_ref[...] * _dis(deg_ref)
        xq_ref[0] = xs[:, :128]
        xq_ref[1] = xs[:, 128:]

    return pl.pallas_call(
        body,
        grid=(GRID,),
        in_specs=[
            pl.BlockSpec((TN, 2), lambda i: (i, 0)),
            pl.BlockSpec((TN, D_IN), lambda i: (i, 0)),
        ],
        out_specs=pl.BlockSpec((2, TN, 128), lambda i: (0, i, 0)),
        out_shape=jax.ShapeDtypeStruct((2, N, 128), jnp.float32),
    )(degt, x)


def _tc_layer1(degt, agg1, xq, W1, b1, W2):
    def body(deg_ref, a_ref, x_ref, w1_ref, b1_ref, w2_ref, y_ref):
        dis = _dis(deg_ref)
        z = jnp.concatenate(
            [a_ref[0] + x_ref[0], a_ref[1] + x_ref[1]], axis=1) * dis
        h = jnp.maximum(
            jnp.dot(z, w1_ref[...], preferred_element_type=jnp.float32)
            + b1_ref[...], 0.0)
        y = jnp.dot(h, w2_ref[...], preferred_element_type=jnp.float32) * dis
        for q in range(4):
            y_ref[q] = y[:, 128 * q:128 * (q + 1)]

    return pl.pallas_call(
        body,
        grid=(GRID,),
        in_specs=[
            pl.BlockSpec((TN, 2), lambda i: (i, 0)),
            pl.BlockSpec((2, TN, 128), lambda i: (0, i, 0)),
            pl.BlockSpec((2, TN, 128), lambda i: (0, i, 0)),
            pl.BlockSpec((D_IN, H), lambda i: (0, 0)),
            pl.BlockSpec((1, H), lambda i: (0, 0)),
            pl.BlockSpec((H, H), lambda i: (0, 0)),
        ],
        out_specs=pl.BlockSpec((4, TN, 128), lambda i: (0, i, 0)),
        out_shape=jax.ShapeDtypeStruct((4, N, 128), jnp.float32),
    )(degt, agg1, xq, W1, b1, W2)


def _tc_layer2(degt, agg2, y2, b2, W3):
    def body(deg_ref, a_ref, y_ref, b2_ref, w3_ref, o_ref):
        dis = _dis(deg_ref)
        s = jnp.concatenate([a_ref[q] + y_ref[q] for q in range(4)], axis=1)
        h = jnp.maximum(s * dis + b2_ref[...], 0.0)
        y = jnp.dot(h, w3_ref[...], preferred_element_type=jnp.float32) * dis
        for q in range(4):
            o_ref[q] = y[:, 128 * q:128 * (q + 1)]

    return pl.pallas_call(
        body,
        grid=(GRID,),
        in_specs=[
            pl.BlockSpec((TN, 2), lambda i: (i, 0)),
            pl.BlockSpec((4, TN, 128), lambda i: (0, i, 0)),
            pl.BlockSpec((4, TN, 128), lambda i: (0, i, 0)),
            pl.BlockSpec((1, H), lambda i: (0, 0)),
            pl.BlockSpec((H, H), lambda i: (0, 0)),
        ],
        out_specs=pl.BlockSpec((4, TN, 128), lambda i: (0, i, 0)),
        out_shape=jax.ShapeDtypeStruct((4, N, 128), jnp.float32),
    )(degt, agg2, y2, b2, W3)


def _tc_final(degt, agg3, y3, b3):
    def body(deg_ref, a_ref, y_ref, b3_ref, o_ref):
        dis = _dis(deg_ref)
        s = jnp.concatenate([a_ref[q] + y_ref[q] for q in range(4)], axis=1)
        o_ref[...] = s * dis + b3_ref[...]

    return pl.pallas_call(
        body,
        grid=(GRID,),
        in_specs=[
            pl.BlockSpec((TN, 2), lambda i: (i, 0)),
            pl.BlockSpec((4, TN, 128), lambda i: (0, i, 0)),
            pl.BlockSpec((4, TN, 128), lambda i: (0, i, 0)),
            pl.BlockSpec((1, H), lambda i: (0, 0)),
        ],
        out_specs=pl.BlockSpec((TN, H), lambda i: (i, 0)),
        out_shape=jax.ShapeDtypeStruct((N, H), jnp.float32),
    )(degt, agg3, y3, b3)


def kernel(x, edge_index, batch, W1, b1, W2, b2, W3, b3):
    src = edge_index[0]
    dst = edge_index[1]
    z_nk = jnp.zeros((N, 128), jnp.float32)
    ones_nk = jnp.ones((N, 128), jnp.float32)

    deg2 = _sc_degree(dst, ones_nk, z_nk)
    degt = deg2[:, :, 0].T

    xq = _tc_pre(degt, x)
    agg1 = _sc_agg(xq, src, dst, z_nk, 2)
    y2 = _tc_layer1(degt, agg1, xq, W1.reshape(D_IN, H), b1.reshape(1, H), W2)
    agg2 = _sc_agg(y2, src, dst, z_nk, 4)
    y3 = _tc_layer2(degt, agg2, y2, b2.reshape(1, H), W3)
    agg3 = _sc_agg(y3, src, dst, z_nk, 4)
    out = _tc_final(degt, agg3, y3, b3.reshape(1, H))
    return (out, out)


# speedup vs baseline: 10.7594x; 1.4170x over previous
import functools

import jax
import jax.numpy as jnp
from jax import lax
from jax.experimental import pallas as pl
from jax.experimental.pallas import tpu as pltpu
from jax.experimental.pallas import tpu_sc as plsc

N = 10000
E = 160000
D_IN = 256
H = 512
K = 128
NCHUNK = E // K
TN = 400
GRID = N // TN

_mesh = plsc.VectorSubcoreMesh(core_axis_name="c", subcore_axis_name="s")


def _zero_rows(z_h, acc, sid):
    @pl.when(sid < 15)
    def _():
        pltpu.sync_copy(z_h.at[pl.ds(sid * 640, 640)],
                        acc.at[pl.ds(sid * 640, 640)])

    @pl.when(sid == 15)
    def _():
        pltpu.sync_copy(z_h.at[pl.ds(9600, 400)], acc.at[pl.ds(9600, 400)])


def _sc_degree(dst, ones_nk, z_nk):
    @functools.partial(
        pl.kernel,
        out_type=jax.ShapeDtypeStruct((2, N, 128), jnp.float32),
        mesh=_mesh,
        scratch_types=[
            pltpu.VMEM((K,), jnp.int32),
            pltpu.VMEM((K, 128), jnp.float32),
            pltpu.VMEM_SHARED((N, 128), jnp.float32),
        ],
    )
    def k(dst_h, ones_h, z_h, out_h, dst_v, msgs_v, acc):
        c = lax.axis_index("c")
        sid = lax.axis_index("s")
        _zero_rows(z_h, acc, sid)
        plsc.subcore_barrier()

        base = c * (NCHUNK // 2)

        def chunk(cid):
            pltpu.sync_copy(dst_h.at[pl.ds(cid * K, K)], dst_v)
            pltpu.sync_copy(ones_h.at[dst_v], msgs_v)
            pltpu.sync_copy(msgs_v, acc.at[dst_v], add=True)

        @pl.loop(0, 39)
        def _(r):
            chunk(base + r * 16 + sid)

        @pl.when(sid == 0)
        def _():
            chunk(base + 624)

        plsc.subcore_barrier()

        @pl.when(sid < 15)
        def _():
            pltpu.sync_copy(acc.at[pl.ds(sid * 640, 640)],
                            out_h.at[c].at[pl.ds(sid * 640, 640)])

        @pl.when(sid == 15)
        def _():
            pltpu.sync_copy(acc.at[pl.ds(9600, 400)],
                            out_h.at[c].at[pl.ds(9600, 400)])

    return k(dst, ones_nk, z_nk)


def _sc_agg(table, src, dst, z_nk, C):
    DEPTH = 3

    @functools.partial(
        pl.kernel,
        out_type=jax.ShapeDtypeStruct((C, N, 128), jnp.float32),
        mesh=_mesh,
        scratch_types=[
            pltpu.VMEM((DEPTH, K), jnp.int32),
            pltpu.VMEM((DEPTH, K), jnp.int32),
            pltpu.VMEM((DEPTH, K, 128), jnp.float32),
            pltpu.VMEM_SHARED((N, 128), jnp.float32),
            pltpu.SemaphoreType.DMA((DEPTH,)),
            pltpu.SemaphoreType.DMA((DEPTH,)),
            pltpu.SemaphoreType.DMA((DEPTH,)),
            pltpu.SemaphoreType.DMA((DEPTH,)),
        ],
    )
    def k(table_h, src_h, dst_h, z_h, out_h, src_v, dst_v, msgs_v, acc,
          sem_i, sem_j, sem_g, sem_s):
        c = lax.axis_index("c")
        sid = lax.axis_index("s")
        for qq in range(C // 2):
            q = c * (C // 2) + qq
            _zero_rows(z_h, acc, sid)
            plsc.subcore_barrier()

            def chunk(cid):
                pltpu.sync_copy(src_h.at[pl.ds(cid * K, K)], src_v.at[0])
                pltpu.sync_copy(dst_h.at[pl.ds(cid * K, K)], dst_v.at[0])
                pltpu.sync_copy(table_h.at[q].at[src_v.at[0]], msgs_v.at[0])
                pltpu.sync_copy(msgs_v.at[0], acc.at[dst_v.at[0]], add=True)

            @pl.loop(0, NCHUNK // 16, step=DEPTH)
            def _(r):
                icps, jcps = [], []
                for i in range(DEPTH):
                    cid = (r + i) * 16 + sid
                    icps.append(pltpu.async_copy(
                        src_h.at[pl.ds(cid * K, K)], src_v.at[i], sem_i.at[i]))
                    jcps.append(pltpu.async_copy(
                        dst_h.at[pl.ds(cid * K, K)], dst_v.at[i], sem_j.at[i]))
                gcps = []
                for i in range(DEPTH):
                    icps[i].wait()
                    gcps.append(pltpu.async_copy(
                        table_h.at[q].at[src_v.at[i]], msgs_v.at[i],
                        sem_g.at[i]))
                scps = []
                for i in range(DEPTH):
                    gcps[i].wait()
                    jcps[i].wait()
                    scps.append(pltpu.async_copy(
                        msgs_v.at[i], acc.at[dst_v.at[i]], sem_s.at[i],
                        add=True))
                for i in range(DEPTH):
                    scps[i].wait()

            @pl.when(sid < NCHUNK - (NCHUNK // 16) * 16)
            def _():
                chunk((NCHUNK // 16) * 16 + sid)

            plsc.subcore_barrier()

            @pl.when(sid < 15)
            def _():
                pltpu.sync_copy(acc.at[pl.ds(sid * 640, 640)],
                                out_h.at[q].at[pl.ds(sid * 640, 640)])

            @pl.when(sid == 15)
            def _():
                pltpu.sync_copy(acc.at[pl.ds(9600, 400)],
                                out_h.at[q].at[pl.ds(9600, 400)])

            if qq + 1 < C // 2:
                plsc.subcore_barrier()

    return k(table, src, dst, z_nk)


def _dis(deg_ref):
    deg = deg_ref[:, 0:1] + deg_ref[:, 1:2] + 1.0
    return lax.rsqrt(deg)


def _tc_pre(degt, x):
    def body(deg_ref, x_ref, xq_ref):
        xs = x_ref[...] * _dis(deg_ref)
        xq_ref[0] = xs[:, :128]
        xq_ref[1] = xs[:, 128:]

    return pl.pallas_call(
        body,
        grid=(GRID,),
        in_specs=[
            pl.BlockSpec((TN, 2), lambda i: (i, 0)),
            pl.BlockSpec((TN, D_IN), lambda i: (i, 0)),
        ],
        out_specs=pl.BlockSpec((2, TN, 128), lambda i: (0, i, 0)),
        out_shape=jax.ShapeDtypeStruct((2, N, 128), jnp.float32),
    )(degt, x)


def _tc_layer1(degt, agg1, xq, W1, b1, W2):
    def body(deg_ref, a_ref, x_ref, w1_ref, b1_ref, w2_ref, y_ref):
        dis = _dis(deg_ref)
        z = jnp.concatenate(
            [a_ref[0] + x_ref[0], a_ref[1] + x_ref[1]], axis=1) * dis
        h = jnp.maximum(
            jnp.dot(z, w1_ref[...], preferred_element_type=jnp.float32)
            + b1_ref[...], 0.0)
        y = jnp.dot(h, w2_ref[...], preferred_element_type=jnp.float32) * dis
        for q in range(4):
            y_ref[q] = y[:, 128 * q:128 * (q + 1)]

    return pl.pallas_call(
        body,
        grid=(GRID,),
        in_specs=[
            pl.BlockSpec((TN, 2), lambda i: (i, 0)),
            pl.BlockSpec((2, TN, 128), lambda i: (0, i, 0)),
            pl.BlockSpec((2, TN, 128), lambda i: (0, i, 0)),
            pl.BlockSpec((D_IN, H), lambda i: (0, 0)),
            pl.BlockSpec((1, H), lambda i: (0, 0)),
            pl.BlockSpec((H, H), lambda i: (0, 0)),
        ],
        out_specs=pl.BlockSpec((4, TN, 128), lambda i: (0, i, 0)),
        out_shape=jax.ShapeDtypeStruct((4, N, 128), jnp.float32),
    )(degt, agg1, xq, W1, b1, W2)


def _tc_layer2(degt, agg2, y2, b2, W3):
    def body(deg_ref, a_ref, y_ref, b2_ref, w3_ref, o_ref):
        dis = _dis(deg_ref)
        s = jnp.concatenate([a_ref[q] + y_ref[q] for q in range(4)], axis=1)
        h = jnp.maximum(s * dis + b2_ref[...], 0.0)
        y = jnp.dot(h, w3_ref[...], preferred_element_type=jnp.float32) * dis
        for q in range(4):
            o_ref[q] = y[:, 128 * q:128 * (q + 1)]

    return pl.pallas_call(
        body,
        grid=(GRID,),
        in_specs=[
            pl.BlockSpec((TN, 2), lambda i: (i, 0)),
            pl.BlockSpec((4, TN, 128), lambda i: (0, i, 0)),
            pl.BlockSpec((4, TN, 128), lambda i: (0, i, 0)),
            pl.BlockSpec((1, H), lambda i: (0, 0)),
            pl.BlockSpec((H, H), lambda i: (0, 0)),
        ],
        out_specs=pl.BlockSpec((4, TN, 128), lambda i: (0, i, 0)),
        out_shape=jax.ShapeDtypeStruct((4, N, 128), jnp.float32),
    )(degt, agg2, y2, b2, W3)


def _tc_final(degt, agg3, y3, b3):
    def body(deg_ref, a_ref, y_ref, b3_ref, o_ref):
        dis = _dis(deg_ref)
        s = jnp.concatenate([a_ref[q] + y_ref[q] for q in range(4)], axis=1)
        o_ref[...] = s * dis + b3_ref[...]

    return pl.pallas_call(
        body,
        grid=(GRID,),
        in_specs=[
            pl.BlockSpec((TN, 2), lambda i: (i, 0)),
            pl.BlockSpec((4, TN, 128), lambda i: (0, i, 0)),
            pl.BlockSpec((4, TN, 128), lambda i: (0, i, 0)),
            pl.BlockSpec((1, H), lambda i: (0, 0)),
        ],
        out_specs=pl.BlockSpec((TN, H), lambda i: (i, 0)),
        out_shape=jax.ShapeDtypeStruct((N, H), jnp.float32),
    )(degt, agg3, y3, b3)


def kernel(x, edge_index, batch, W1, b1, W2, b2, W3, b3):
    src = edge_index[0]
    dst = edge_index[1]
    z_nk = jnp.zeros((N, 128), jnp.float32)
    ones_nk = jnp.ones((N, 128), jnp.float32)

    deg2 = _sc_degree(dst, ones_nk, z_nk)
    degt = deg2[:, :, 0].T

    xq = _tc_pre(degt, x)
    agg1 = _sc_agg(xq, src, dst, z_nk, 2)
    y2 = _tc_layer1(degt, agg1, xq, W1.reshape(D_IN, H), b1.reshape(1, H), W2)
    agg2 = _sc_agg(y2, src, dst, z_nk, 4)
    y3 = _tc_layer2(degt, agg2, y2, b2.reshape(1, H), W3)
    agg3 = _sc_agg(y3, src, dst, z_nk, 4)
    out = _tc_final(degt, agg3, y3, b3.reshape(1, H))
    return (out, out)


# speedup vs baseline: 11.0425x; 1.0263x over previous
import functools

import jax
import jax.numpy as jnp
from jax import lax
from jax.experimental import pallas as pl
from jax.experimental.pallas import tpu as pltpu
from jax.experimental.pallas import tpu_sc as plsc

N = 10000
E = 160000
D_IN = 256
H = 512
K = 128
NCHUNK = E // K
TN = 400
GRID = N // TN

_mesh = plsc.VectorSubcoreMesh(core_axis_name="c", subcore_axis_name="s")


def _zero_rows(z_h, acc, sid):
    @pl.when(sid < 15)
    def _():
        pltpu.sync_copy(z_h.at[pl.ds(sid * 640, 640)],
                        acc.at[pl.ds(sid * 640, 640)])

    @pl.when(sid == 15)
    def _():
        pltpu.sync_copy(z_h.at[pl.ds(9600, 400)], acc.at[pl.ds(9600, 400)])


def _sc_degree(dst, ones_nk, z_nk):
    DEPTH = 3

    @functools.partial(
        pl.kernel,
        out_type=jax.ShapeDtypeStruct((2, N, 128), jnp.float32),
        mesh=_mesh,
        scratch_types=[
            pltpu.VMEM((DEPTH, K), jnp.int32),
            pltpu.VMEM((DEPTH, K, 128), jnp.float32),
            pltpu.VMEM_SHARED((N, 128), jnp.float32),
            pltpu.SemaphoreType.DMA((DEPTH,)),
            pltpu.SemaphoreType.DMA((DEPTH,)),
            pltpu.SemaphoreType.DMA((DEPTH,)),
        ],
    )
    def k(dst_h, ones_h, z_h, out_h, dst_v, msgs_v, acc, sem_j, sem_g, sem_s):
        c = lax.axis_index("c")
        sid = lax.axis_index("s")
        _zero_rows(z_h, acc, sid)
        plsc.subcore_barrier()

        base = c * (NCHUNK // 2)

        @pl.loop(0, 39, step=DEPTH)
        def _(r):
            jcps = []
            for i in range(DEPTH):
                cid = base + (r + i) * 16 + sid
                jcps.append(pltpu.async_copy(
                    dst_h.at[pl.ds(cid * K, K)], dst_v.at[i], sem_j.at[i]))
            gcps = []
            for i in range(DEPTH):
                jcps[i].wait()
                gcps.append(pltpu.async_copy(
                    ones_h.at[dst_v.at[i]], msgs_v.at[i], sem_g.at[i]))
            scps = []
            for i in range(DEPTH):
                gcps[i].wait()
                scps.append(pltpu.async_copy(
                    msgs_v.at[i], acc.at[dst_v.at[i]], sem_s.at[i], add=True))
            for i in range(DEPTH):
                scps[i].wait()

        @pl.when(sid == 0)
        def _():
            cid = base + 624
            pltpu.sync_copy(dst_h.at[pl.ds(cid * K, K)], dst_v.at[0])
            pltpu.sync_copy(ones_h.at[dst_v.at[0]], msgs_v.at[0])
            pltpu.sync_copy(msgs_v.at[0], acc.at[dst_v.at[0]], add=True)

        plsc.subcore_barrier()

        @pl.when(sid < 15)
        def _():
            pltpu.sync_copy(acc.at[pl.ds(sid * 640, 640)],
                            out_h.at[c].at[pl.ds(sid * 640, 640)])

        @pl.when(sid == 15)
        def _():
            pltpu.sync_copy(acc.at[pl.ds(9600, 400)],
                            out_h.at[c].at[pl.ds(9600, 400)])

    return k(dst, ones_nk, z_nk)


def _sc_agg(table, src, dst, z_nk, C):
    DEPTH = 3

    @functools.partial(
        pl.kernel,
        out_type=jax.ShapeDtypeStruct((C, N, 128), jnp.float32),
        mesh=_mesh,
        scratch_types=[
            pltpu.VMEM((DEPTH, K), jnp.int32),
            pltpu.VMEM((DEPTH, K), jnp.int32),
            pltpu.VMEM((DEPTH, K, 128), jnp.float32),
            pltpu.VMEM_SHARED((N, 128), jnp.float32),
            pltpu.SemaphoreType.DMA((DEPTH,)),
            pltpu.SemaphoreType.DMA((DEPTH,)),
            pltpu.SemaphoreType.DMA((DEPTH,)),
            pltpu.SemaphoreType.DMA((DEPTH,)),
        ],
    )
    def k(table_h, src_h, dst_h, z_h, out_h, src_v, dst_v, msgs_v, acc,
          sem_i, sem_j, sem_g, sem_s):
        c = lax.axis_index("c")
        sid = lax.axis_index("s")
        for qq in range(C // 2):
            q = c * (C // 2) + qq
            _zero_rows(z_h, acc, sid)
            plsc.subcore_barrier()

            def chunk(cid):
                pltpu.sync_copy(src_h.at[pl.ds(cid * K, K)], src_v.at[0])
                pltpu.sync_copy(dst_h.at[pl.ds(cid * K, K)], dst_v.at[0])
                pltpu.sync_copy(table_h.at[q].at[src_v.at[0]], msgs_v.at[0])
                pltpu.sync_copy(msgs_v.at[0], acc.at[dst_v.at[0]], add=True)

            @pl.loop(0, NCHUNK // 16, step=DEPTH)
            def _(r):
                icps, jcps = [], []
                for i in range(DEPTH):
                    cid = (r + i) * 16 + sid
                    icps.append(pltpu.async_copy(
                        src_h.at[pl.ds(cid * K, K)], src_v.at[i], sem_i.at[i]))
                    jcps.append(pltpu.async_copy(
                        dst_h.at[pl.ds(cid * K, K)], dst_v.at[i], sem_j.at[i]))
                gcps = []
                for i in range(DEPTH):
                    icps[i].wait()
                    gcps.append(pltpu.async_copy(
                        table_h.at[q].at[src_v.at[i]], msgs_v.at[i],
                        sem_g.at[i]))
                scps = []
                for i in range(DEPTH):
                    gcps[i].wait()
                    jcps[i].wait()
                    scps.append(pltpu.async_copy(
                        msgs_v.at[i], acc.at[dst_v.at[i]], sem_s.at[i],
                        add=True))
                for i in range(DEPTH):
                    scps[i].wait()

            @pl.when(sid < NCHUNK - (NCHUNK // 16) * 16)
            def _():
                chunk((NCHUNK // 16) * 16 + sid)

            plsc.subcore_barrier()

            @pl.when(sid < 15)
            def _():
                pltpu.sync_copy(acc.at[pl.ds(sid * 640, 640)],
                                out_h.at[q].at[pl.ds(sid * 640, 640)])

            @pl.when(sid == 15)
            def _():
                pltpu.sync_copy(acc.at[pl.ds(9600, 400)],
                                out_h.at[q].at[pl.ds(9600, 400)])

            if qq + 1 < C // 2:
                plsc.subcore_barrier()

    return k(table, src, dst, z_nk)


def _dis(deg_ref):
    deg = deg_ref[:, 0:1] + deg_ref[:, 1:2] + 1.0
    return lax.rsqrt(deg)


def _tc_pre(degt, x):
    def body(deg_ref, x_ref, xq_ref):
        xs = x_ref[...] * _dis(deg_ref)
        xq_ref[0] = xs[:, :128]
        xq_ref[1] = xs[:, 128:]

    return pl.pallas_call(
        body,
        grid=(GRID,),
        in_specs=[
            pl.BlockSpec((TN, 2), lambda i: (i, 0)),
            pl.BlockSpec((TN, D_IN), lambda i: (i, 0)),
        ],
        out_specs=pl.BlockSpec((2, TN, 128), lambda i: (0, i, 0)),
        out_shape=jax.ShapeDtypeStruct((2, N, 128), jnp.float32),
    )(degt, x)


def _tc_layer1(degt, agg1, xq, W1, b1, W2):
    def body(deg_ref, a_ref, x_ref, w1_ref, b1_ref, w2_ref, y_ref):
        dis = _dis(deg_ref)
        z = jnp.concatenate(
            [a_ref[0] + x_ref[0], a_ref[1] + x_ref[1]], axis=1) * dis
        h = jnp.maximum(
            jnp.dot(z, w1_ref[...], preferred_element_type=jnp.float32)
            + b1_ref[...], 0.0)
        y = jnp.dot(h, w2_ref[...], preferred_element_type=jnp.float32) * dis
        for q in range(4):
            y_ref[q] = y[:, 128 * q:128 * (q + 1)]

    return pl.pallas_call(
        body,
        grid=(GRID,),
        in_specs=[
            pl.BlockSpec((TN, 2), lambda i: (i, 0)),
            pl.BlockSpec((2, TN, 128), lambda i: (0, i, 0)),
            pl.BlockSpec((2, TN, 128), lambda i: (0, i, 0)),
            pl.BlockSpec((D_IN, H), lambda i: (0, 0)),
            pl.BlockSpec((1, H), lambda i: (0, 0)),
            pl.BlockSpec((H, H), lambda i: (0, 0)),
        ],
        out_specs=pl.BlockSpec((4, TN, 128), lambda i: (0, i, 0)),
        out_shape=jax.ShapeDtypeStruct((4, N, 128), jnp.float32),
    )(degt, agg1, xq, W1, b1, W2)


def _tc_layer2(degt, agg2, y2, b2, W3):
    def body(deg_ref, a_ref, y_ref, b2_ref, w3_ref, o_ref):
        dis = _dis(deg_ref)
        s = jnp.concatenate([a_ref[q] + y_ref[q] for q in range(4)], axis=1)
        h = jnp.maximum(s * dis + b2_ref[...], 0.0)
        y = jnp.dot(h, w3_ref[...], preferred_element_type=jnp.float32) * dis
        for q in range(4):
            o_ref[q] = y[:, 128 * q:128 * (q + 1)]

    return pl.pallas_call(
        body,
        grid=(GRID,),
        in_specs=[
            pl.BlockSpec((TN, 2), lambda i: (i, 0)),
            pl.BlockSpec((4, TN, 128), lambda i: (0, i, 0)),
            pl.BlockSpec((4, TN, 128), lambda i: (0, i, 0)),
            pl.BlockSpec((1, H), lambda i: (0, 0)),
            pl.BlockSpec((H, H), lambda i: (0, 0)),
        ],
        out_specs=pl.BlockSpec((4, TN, 128), lambda i: (0, i, 0)),
        out_shape=jax.ShapeDtypeStruct((4, N, 128), jnp.float32),
    )(degt, agg2, y2, b2, W3)


def _tc_final(degt, agg3, y3, b3):
    def body(deg_ref, a_ref, y_ref, b3_ref, o_ref):
        dis = _dis(deg_ref)
        s = jnp.concatenate([a_ref[q] + y_ref[q] for q in range(4)], axis=1)
        o_ref[...] = s * dis + b3_ref[...]

    return pl.pallas_call(
        body,
        grid=(GRID,),
        in_specs=[
            pl.BlockSpec((TN, 2), lambda i: (i, 0)),
            pl.BlockSpec((4, TN, 128), lambda i: (0, i, 0)),
            pl.BlockSpec((4, TN, 128), lambda i: (0, i, 0)),
            pl.BlockSpec((1, H), lambda i: (0, 0)),
        ],
        out_specs=pl.BlockSpec((TN, H), lambda i: (i, 0)),
        out_shape=jax.ShapeDtypeStruct((N, H), jnp.float32),
    )(degt, agg3, y3, b3)


def kernel(x, edge_index, batch, W1, b1, W2, b2, W3, b3):
    src = edge_index[0]
    dst = edge_index[1]
    z_nk = jnp.zeros((N, 128), jnp.float32)
    ones_nk = jnp.ones((N, 128), jnp.float32)

    deg2 = _sc_degree(dst, ones_nk, z_nk)
    degt = deg2[:, :, 0].T

    xq = _tc_pre(degt, x)
    agg1 = _sc_agg(xq, src, dst, z_nk, 2)
    y2 = _tc_layer1(degt, agg1, xq, W1.reshape(D_IN, H), b1.reshape(1, H), W2)
    agg2 = _sc_agg(y2, src, dst, z_nk, 4)
    y3 = _tc_layer2(degt, agg2, y2, b2.reshape(1, H), W3)
    agg3 = _sc_agg(y3, src, dst, z_nk, 4)
    out = _tc_final(degt, agg3, y3, b3.reshape(1, H))
    return (out, out)


# speedup vs baseline: 11.8859x; 1.0764x over previous
import functools

import jax
import jax.numpy as jnp
from jax import lax
from jax.experimental import pallas as pl
from jax.experimental.pallas import tpu as pltpu
from jax.experimental.pallas import tpu_sc as plsc

N = 10000
E = 160000
D_IN = 256
H = 512
K = 128
NCHUNK = E // K
TN = 400
GRID = N // TN

_mesh = plsc.VectorSubcoreMesh(core_axis_name="c", subcore_axis_name="s")


def _zero_rows(z_h, acc, sid):
    @pl.when(sid < 15)
    def _():
        pltpu.sync_copy(z_h.at[pl.ds(sid * 640, 640)],
                        acc.at[pl.ds(sid * 640, 640)])

    @pl.when(sid == 15)
    def _():
        pltpu.sync_copy(z_h.at[pl.ds(9600, 400)], acc.at[pl.ds(9600, 400)])


def _sc_degree(dst, ones_nk, z_nk):
    DEPTH = 3

    @functools.partial(
        pl.kernel,
        out_type=jax.ShapeDtypeStruct((2, N, 128), jnp.float32),
        mesh=_mesh,
        scratch_types=[
            pltpu.VMEM((DEPTH, K), jnp.int32),
            pltpu.VMEM((DEPTH, K, 128), jnp.float32),
            pltpu.VMEM_SHARED((N, 128), jnp.float32),
            pltpu.SemaphoreType.DMA((DEPTH,)),
            pltpu.SemaphoreType.DMA((DEPTH,)),
            pltpu.SemaphoreType.DMA((DEPTH,)),
        ],
    )
    def k(dst_h, ones_h, z_h, out_h, dst_v, msgs_v, acc, sem_j, sem_g, sem_s):
        c = lax.axis_index("c")
        sid = lax.axis_index("s")
        _zero_rows(z_h, acc, sid)
        plsc.subcore_barrier()

        base = c * (NCHUNK // 2)

        @pl.loop(0, 39, step=DEPTH)
        def _(r):
            jcps = []
            for i in range(DEPTH):
                cid = base + (r + i) * 16 + sid
                jcps.append(pltpu.async_copy(
                    dst_h.at[pl.ds(cid * K, K)], dst_v.at[i], sem_j.at[i]))
            gcps = []
            for i in range(DEPTH):
                jcps[i].wait()
                gcps.append(pltpu.async_copy(
                    ones_h.at[dst_v.at[i]], msgs_v.at[i], sem_g.at[i]))
            scps = []
            for i in range(DEPTH):
                gcps[i].wait()
                scps.append(pltpu.async_copy(
                    msgs_v.at[i], acc.at[dst_v.at[i]], sem_s.at[i], add=True))
            for i in range(DEPTH):
                scps[i].wait()

        @pl.when(sid == 0)
        def _():
            cid = base + 624
            pltpu.sync_copy(dst_h.at[pl.ds(cid * K, K)], dst_v.at[0])
            pltpu.sync_copy(ones_h.at[dst_v.at[0]], msgs_v.at[0])
            pltpu.sync_copy(msgs_v.at[0], acc.at[dst_v.at[0]], add=True)

        plsc.subcore_barrier()

        @pl.when(sid < 15)
        def _():
            pltpu.sync_copy(acc.at[pl.ds(sid * 640, 640)],
                            out_h.at[c].at[pl.ds(sid * 640, 640)])

        @pl.when(sid == 15)
        def _():
            pltpu.sync_copy(acc.at[pl.ds(9600, 400)],
                            out_h.at[c].at[pl.ds(9600, 400)])

    return k(dst, ones_nk, z_nk)


def _sc_agg(table, src, dst, z_nk, C):
    DEPTH = 3

    @functools.partial(
        pl.kernel,
        out_type=jax.ShapeDtypeStruct((C, N, 128), jnp.float32),
        mesh=_mesh,
        scratch_types=[
            pltpu.VMEM((DEPTH, K), jnp.int32),
            pltpu.VMEM((DEPTH, K), jnp.int32),
            pltpu.VMEM((DEPTH, K, 128), jnp.float32),
            pltpu.VMEM_SHARED((N, 128), jnp.float32),
            pltpu.SemaphoreType.DMA((DEPTH,)),
            pltpu.SemaphoreType.DMA((DEPTH,)),
            pltpu.SemaphoreType.DMA((DEPTH,)),
            pltpu.SemaphoreType.DMA((DEPTH,)),
        ],
    )
    def k(table_h, src_h, dst_h, z_h, out_h, src_v, dst_v, msgs_v, acc,
          sem_i, sem_j, sem_g, sem_s):
        c = lax.axis_index("c")
        sid = lax.axis_index("s")
        for qq in range(C // 2):
            q = c * (C // 2) + qq
            _zero_rows(z_h, acc, sid)
            plsc.subcore_barrier()

            def chunk(cid):
                pltpu.sync_copy(src_h.at[pl.ds(cid * K, K)], src_v.at[0])
                pltpu.sync_copy(dst_h.at[pl.ds(cid * K, K)], dst_v.at[0])
                pltpu.sync_copy(table_h.at[q].at[src_v.at[0]], msgs_v.at[0])
                pltpu.sync_copy(msgs_v.at[0], acc.at[dst_v.at[0]], add=True)

            def batch(r, first):
                icps, jcps = [], []
                for i in range(DEPTH):
                    if not first:
                        pltpu.make_async_copy(
                            msgs_v.at[i], acc.at[dst_v.at[i]],
                            sem_s.at[i]).wait()
                    cid = (r + i) * 16 + sid
                    icps.append(pltpu.async_copy(
                        src_h.at[pl.ds(cid * K, K)], src_v.at[i], sem_i.at[i]))
                    jcps.append(pltpu.async_copy(
                        dst_h.at[pl.ds(cid * K, K)], dst_v.at[i], sem_j.at[i]))
                gcps = []
                for i in range(DEPTH):
                    icps[i].wait()
                    gcps.append(pltpu.async_copy(
                        table_h.at[q].at[src_v.at[i]], msgs_v.at[i],
                        sem_g.at[i]))
                for i in range(DEPTH):
                    gcps[i].wait()
                    jcps[i].wait()
                    pltpu.async_copy(
                        msgs_v.at[i], acc.at[dst_v.at[i]], sem_s.at[i],
                        add=True)

            batch(0, first=True)

            @pl.loop(DEPTH, NCHUNK // 16, step=DEPTH)
            def _(r):
                batch(r, first=False)

            for i in range(DEPTH):
                pltpu.make_async_copy(
                    msgs_v.at[i], acc.at[dst_v.at[i]], sem_s.at[i]).wait()

            @pl.when(sid < NCHUNK - (NCHUNK // 16) * 16)
            def _():
                chunk((NCHUNK // 16) * 16 + sid)

            plsc.subcore_barrier()

            @pl.when(sid < 15)
            def _():
                pltpu.sync_copy(acc.at[pl.ds(sid * 640, 640)],
                                out_h.at[q].at[pl.ds(sid * 640, 640)])

            @pl.when(sid == 15)
            def _():
                pltpu.sync_copy(acc.at[pl.ds(9600, 400)],
                                out_h.at[q].at[pl.ds(9600, 400)])

            if qq + 1 < C // 2:
                plsc.subcore_barrier()

    return k(table, src, dst, z_nk)


def _dis(deg_ref):
    deg = deg_ref[:, 0:1] + deg_ref[:, 1:2] + 1.0
    return lax.rsqrt(deg)


def _tc_pre(degt, x):
    def body(deg_ref, x_ref, xq_ref):
        xs = x_ref[...] * _dis(deg_ref)
        xq_ref[0] = xs[:, :128]
        xq_ref[1] = xs[:, 128:]

    return pl.pallas_call(
        body,
        grid=(GRID,),
        in_specs=[
            pl.BlockSpec((TN, 2), lambda i: (i, 0)),
            pl.BlockSpec((TN, D_IN), lambda i: (i, 0)),
        ],
        out_specs=pl.BlockSpec((2, TN, 128), lambda i: (0, i, 0)),
        out_shape=jax.ShapeDtypeStruct((2, N, 128), jnp.float32),
    )(degt, x)


def _tc_layer1(degt, agg1, xq, W1, b1, W2):
    def body(deg_ref, a_ref, x_ref, w1_ref, b1_ref, w2_ref, y_ref):
        dis = _dis(deg_ref)
        z = jnp.concatenate(
            [a_ref[0] + x_ref[0], a_ref[1] + x_ref[1]], axis=1) * dis
        h = jnp.maximum(
            jnp.dot(z, w1_ref[...], preferred_element_type=jnp.float32)
            + b1_ref[...], 0.0)
        y = jnp.dot(h, w2_ref[...], preferred_element_type=jnp.float32) * dis
        for q in range(4):
            y_ref[q] = y[:, 128 * q:128 * (q + 1)]

    return pl.pallas_call(
        body,
        grid=(GRID,),
        in_specs=[
            pl.BlockSpec((TN, 2), lambda i: (i, 0)),
            pl.BlockSpec((2, TN, 128), lambda i: (0, i, 0)),
            pl.BlockSpec((2, TN, 128), lambda i: (0, i, 0)),
            pl.BlockSpec((D_IN, H), lambda i: (0, 0)),
            pl.BlockSpec((1, H), lambda i: (0, 0)),
            pl.BlockSpec((H, H), lambda i: (0, 0)),
        ],
        out_specs=pl.BlockSpec((4, TN, 128), lambda i: (0, i, 0)),
        out_shape=jax.ShapeDtypeStruct((4, N, 128), jnp.float32),
    )(degt, agg1, xq, W1, b1, W2)


def _tc_layer2(degt, agg2, y2, b2, W3):
    def body(deg_ref, a_ref, y_ref, b2_ref, w3_ref, o_ref):
        dis = _dis(deg_ref)
        s = jnp.concatenate([a_ref[q] + y_ref[q] for q in range(4)], axis=1)
        h = jnp.maximum(s * dis + b2_ref[...], 0.0)
        y = jnp.dot(h, w3_ref[...], preferred_element_type=jnp.float32) * dis
        for q in range(4):
            o_ref[q] = y[:, 128 * q:128 * (q + 1)]

    return pl.pallas_call(
        body,
        grid=(GRID,),
        in_specs=[
            pl.BlockSpec((TN, 2), lambda i: (i, 0)),
            pl.BlockSpec((4, TN, 128), lambda i: (0, i, 0)),
            pl.BlockSpec((4, TN, 128), lambda i: (0, i, 0)),
            pl.BlockSpec((1, H), lambda i: (0, 0)),
            pl.BlockSpec((H, H), lambda i: (0, 0)),
        ],
        out_specs=pl.BlockSpec((4, TN, 128), lambda i: (0, i, 0)),
        out_shape=jax.ShapeDtypeStruct((4, N, 128), jnp.float32),
    )(degt, agg2, y2, b2, W3)


def _tc_final(degt, agg3, y3, b3):
    def body(deg_ref, a_ref, y_ref, b3_ref, o_ref):
        dis = _dis(deg_ref)
        s = jnp.concatenate([a_ref[q] + y_ref[q] for q in range(4)], axis=1)
        o_ref[...] = s * dis + b3_ref[...]

    return pl.pallas_call(
        body,
        grid=(GRID,),
        in_specs=[
            pl.BlockSpec((TN, 2), lambda i: (i, 0)),
            pl.BlockSpec((4, TN, 128), lambda i: (0, i, 0)),
            pl.BlockSpec((4, TN, 128), lambda i: (0, i, 0)),
            pl.BlockSpec((1, H), lambda i: (0, 0)),
        ],
        out_specs=pl.BlockSpec((TN, H), lambda i: (i, 0)),
        out_shape=jax.ShapeDtypeStruct((N, H), jnp.float32),
    )(degt, agg3, y3, b3)


def kernel(x, edge_index, batch, W1, b1, W2, b2, W3, b3):
    src = edge_index[0]
    dst = edge_index[1]
    z_nk = jnp.zeros((N, 128), jnp.float32)
    ones_nk = jnp.ones((N, 128), jnp.float32)

    deg2 = _sc_degree(dst, ones_nk, z_nk)
    degt = deg2[:, :, 0].T

    xq = _tc_pre(degt, x)
    agg1 = _sc_agg(xq, src, dst, z_nk, 2)
    y2 = _tc_layer1(degt, agg1, xq, W1.reshape(D_IN, H), b1.reshape(1, H), W2)
    agg2 = _sc_agg(y2, src, dst, z_nk, 4)
    y3 = _tc_layer2(degt, agg2, y2, b2.reshape(1, H), W3)
    agg3 = _sc_agg(y3, src, dst, z_nk, 4)
    out = _tc_final(degt, agg3, y3, b3.reshape(1, H))
    return (out, out)


# speedup vs baseline: 13.5359x; 1.1388x over previous
import functools

import jax
import jax.numpy as jnp
from jax import lax
from jax.experimental import pallas as pl
from jax.experimental.pallas import tpu as pltpu
from jax.experimental.pallas import tpu_sc as plsc

N = 10000
E = 160000
D_IN = 256
H = 512
K = 128
NCHUNK = E // K
TN = 400
GRID = N // TN

_mesh = plsc.VectorSubcoreMesh(core_axis_name="c", subcore_axis_name="s")


def _zero_rows(z_h, acc, sid):
    @pl.when(sid < 15)
    def _():
        pltpu.sync_copy(z_h.at[pl.ds(sid * 640, 640)],
                        acc.at[pl.ds(sid * 640, 640)])

    @pl.when(sid == 15)
    def _():
        pltpu.sync_copy(z_h.at[pl.ds(9600, 400)], acc.at[pl.ds(9600, 400)])


def _sc_degree(dst, ones_kk):
    DEPTH = 3

    @functools.partial(
        pl.kernel,
        out_type=jax.ShapeDtypeStruct((2, N, 128), jnp.float32),
        mesh=_mesh,
        scratch_types=[
            pltpu.VMEM((DEPTH, K), jnp.int32),
            pltpu.VMEM((K, 128), jnp.float32),
            pltpu.VMEM((K, 128), jnp.float32),
            pltpu.VMEM_SHARED((N, 128), jnp.float32),
            pltpu.SemaphoreType.DMA((DEPTH,)),
            pltpu.SemaphoreType.DMA((DEPTH,)),
            pltpu.SemaphoreType.DMA,
        ],
    )
    def k(dst_h, ones_h, out_h, dst_v, ones_v, zero_v, acc,
          sem_j, sem_s, sem_z):
        c = lax.axis_index("c")
        sid = lax.axis_index("s")
        pltpu.sync_copy(ones_h, ones_v)

        @pl.loop(0, K)
        def _(i):
            @pl.loop(0, 128, step=16)
            def _(j):
                zero_v[i, pl.ds(j, 16)] = jnp.zeros((16,), jnp.float32)

        @pl.when(sid < 15)
        def _():
            zc = [pltpu.async_copy(
                zero_v, acc.at[pl.ds(sid * 640 + t * 128, 128)], sem_z)
                for t in range(5)]
            for z in zc:
                z.wait()

        @pl.when(sid == 15)
        def _():
            zc = [pltpu.async_copy(
                zero_v, acc.at[pl.ds(9600 + t * 128, 128)], sem_z)
                for t in range(3)]
            zc.append(pltpu.async_copy(
                zero_v.at[pl.ds(0, 16)], acc.at[pl.ds(9984, 16)], sem_z))
            for z in zc:
                z.wait()

        plsc.subcore_barrier()

        base = c * (NCHUNK // 2)

        def batch(r, first):
            jcps = []
            for i in range(DEPTH):
                if not first:
                    pltpu.make_async_copy(
                        ones_v, acc.at[dst_v.at[i]], sem_s.at[i]).wait()
                cid = base + (r + i) * 16 + sid
                jcps.append(pltpu.async_copy(
                    dst_h.at[pl.ds(cid * K, K)], dst_v.at[i], sem_j.at[i]))
            for i in range(DEPTH):
                jcps[i].wait()
                pltpu.async_copy(
                    ones_v, acc.at[dst_v.at[i]], sem_s.at[i], add=True)

        batch(0, first=True)

        @pl.loop(DEPTH, 39, step=DEPTH)
        def _(r):
            batch(r, first=False)

        for i in range(DEPTH):
            pltpu.make_async_copy(
                ones_v, acc.at[dst_v.at[i]], sem_s.at[i]).wait()

        @pl.when(sid == 0)
        def _():
            cid = base + 624
            pltpu.sync_copy(dst_h.at[pl.ds(cid * K, K)], dst_v.at[0])
            pltpu.sync_copy(ones_v, acc.at[dst_v.at[0]], add=True)

        plsc.subcore_barrier()

        @pl.when(sid < 15)
        def _():
            pltpu.sync_copy(acc.at[pl.ds(sid * 640, 640)],
                            out_h.at[c].at[pl.ds(sid * 640, 640)])

        @pl.when(sid == 15)
        def _():
            pltpu.sync_copy(acc.at[pl.ds(9600, 400)],
                            out_h.at[c].at[pl.ds(9600, 400)])

    return k(dst, ones_kk)


def _sc_agg(table, src, dst, z_nk, C):
    DEPTH = 3

    @functools.partial(
        pl.kernel,
        out_type=jax.ShapeDtypeStruct((C, N, 128), jnp.float32),
        mesh=_mesh,
        scratch_types=[
            pltpu.VMEM((DEPTH, K), jnp.int32),
            pltpu.VMEM((DEPTH, K), jnp.int32),
            pltpu.VMEM((DEPTH, K, 128), jnp.float32),
            pltpu.VMEM_SHARED((N, 128), jnp.float32),
            pltpu.SemaphoreType.DMA((DEPTH,)),
            pltpu.SemaphoreType.DMA((DEPTH,)),
            pltpu.SemaphoreType.DMA((DEPTH,)),
            pltpu.SemaphoreType.DMA((DEPTH,)),
            pltpu.SemaphoreType.DMA,
        ],
    )
    def k(table_h, src_h, dst_h, z_h, out_h, src_v, dst_v, msgs_v, acc,
          sem_i, sem_j, sem_g, sem_s, sem_z):
        c = lax.axis_index("c")
        sid = lax.axis_index("s")
        for qq in range(C // 2):
            q = c * (C // 2) + qq

            @pl.when(sid < 15)
            def _():
                pltpu.async_copy(z_h.at[pl.ds(sid * 640, 640)],
                                 acc.at[pl.ds(sid * 640, 640)], sem_z)

            @pl.when(sid == 15)
            def _():
                pltpu.async_copy(z_h.at[pl.ds(9600, 400)],
                                 acc.at[pl.ds(9600, 400)], sem_z)

            def chunk(cid):
                pltpu.sync_copy(src_h.at[pl.ds(cid * K, K)], src_v.at[0])
                pltpu.sync_copy(dst_h.at[pl.ds(cid * K, K)], dst_v.at[0])
                pltpu.sync_copy(table_h.at[q].at[src_v.at[0]], msgs_v.at[0])
                pltpu.sync_copy(msgs_v.at[0], acc.at[dst_v.at[0]], add=True)

            def issue_idx(r, ring):
                icps, jcps = [], []
                for i in range(DEPTH):
                    if ring:
                        pltpu.make_async_copy(
                            msgs_v.at[i], acc.at[dst_v.at[i]],
                            sem_s.at[i]).wait()
                    cid = (r + i) * 16 + sid
                    icps.append(pltpu.async_copy(
                        src_h.at[pl.ds(cid * K, K)], src_v.at[i], sem_i.at[i]))
                    jcps.append(pltpu.async_copy(
                        dst_h.at[pl.ds(cid * K, K)], dst_v.at[i], sem_j.at[i]))
                return icps, jcps

            def issue_gathers(icps):
                gcps = []
                for i in range(DEPTH):
                    icps[i].wait()
                    gcps.append(pltpu.async_copy(
                        table_h.at[q].at[src_v.at[i]], msgs_v.at[i],
                        sem_g.at[i]))
                return gcps

            def issue_scatters(gcps, jcps):
                for i in range(DEPTH):
                    gcps[i].wait()
                    jcps[i].wait()
                    pltpu.async_copy(
                        msgs_v.at[i], acc.at[dst_v.at[i]], sem_s.at[i],
                        add=True)

            icps0, jcps0 = issue_idx(0, ring=False)
            gcps0 = issue_gathers(icps0)

            @pl.when(sid < 15)
            def _():
                pltpu.make_async_copy(z_h.at[pl.ds(sid * 640, 640)],
                                      acc.at[pl.ds(sid * 640, 640)],
                                      sem_z).wait()

            @pl.when(sid == 15)
            def _():
                pltpu.make_async_copy(z_h.at[pl.ds(9600, 400)],
                                      acc.at[pl.ds(9600, 400)], sem_z).wait()

            plsc.subcore_barrier()
            issue_scatters(gcps0, jcps0)

            @pl.loop(DEPTH, NCHUNK // 16, step=DEPTH)
            def _(r):
                icps, jcps = issue_idx(r, ring=True)
                issue_scatters(issue_gathers(icps), jcps)

            for i in range(DEPTH):
                pltpu.make_async_copy(
                    msgs_v.at[i], acc.at[dst_v.at[i]], sem_s.at[i]).wait()

            @pl.when(sid < NCHUNK - (NCHUNK // 16) * 16)
            def _():
                chunk((NCHUNK // 16) * 16 + sid)

            plsc.subcore_barrier()

            @pl.when(sid < 15)
            def _():
                pltpu.sync_copy(acc.at[pl.ds(sid * 640, 640)],
                                out_h.at[q].at[pl.ds(sid * 640, 640)])

            @pl.when(sid == 15)
            def _():
                pltpu.sync_copy(acc.at[pl.ds(9600, 400)],
                                out_h.at[q].at[pl.ds(9600, 400)])

            if qq + 1 < C // 2:
                plsc.subcore_barrier()

    return k(table, src, dst, z_nk)


def _dis(deg_ref):
    deg = deg_ref[0, :, 0:1] + deg_ref[1, :, 0:1] + 1.0
    return lax.rsqrt(deg)


def _tc_pre(deg2, x):
    def body(deg_ref, x_ref, xq_ref):
        xs = x_ref[...] * _dis(deg_ref)
        xq_ref[0] = xs[:, :128]
        xq_ref[1] = xs[:, 128:]

    return pl.pallas_call(
        body,
        grid=(GRID,),
        in_specs=[
            pl.BlockSpec((2, TN, 128), lambda i: (0, i, 0)),
            pl.BlockSpec((TN, D_IN), lambda i: (i, 0)),
        ],
        out_specs=pl.BlockSpec((2, TN, 128), lambda i: (0, i, 0)),
        out_shape=jax.ShapeDtypeStruct((2, N, 128), jnp.float32),
    )(deg2, x)


def _tc_layer1(deg2, agg1, xq, W1, b1, W2):
    def body(deg_ref, a_ref, x_ref, w1_ref, b1_ref, w2_ref, y_ref):
        dis = _dis(deg_ref)
        z = jnp.concatenate(
            [a_ref[0] + x_ref[0], a_ref[1] + x_ref[1]], axis=1) * dis
        h = jnp.maximum(
            jnp.dot(z, w1_ref[...], preferred_element_type=jnp.float32)
            + b1_ref[...], 0.0)
        y = jnp.dot(h, w2_ref[...], preferred_element_type=jnp.float32) * dis
        for q in range(4):
            y_ref[q] = y[:, 128 * q:128 * (q + 1)]

    return pl.pallas_call(
        body,
        grid=(GRID,),
        in_specs=[
            pl.BlockSpec((2, TN, 128), lambda i: (0, i, 0)),
            pl.BlockSpec((2, TN, 128), lambda i: (0, i, 0)),
            pl.BlockSpec((2, TN, 128), lambda i: (0, i, 0)),
            pl.BlockSpec((D_IN, H), lambda i: (0, 0)),
            pl.BlockSpec((1, H), lambda i: (0, 0)),
            pl.BlockSpec((H, H), lambda i: (0, 0)),
        ],
        out_specs=pl.BlockSpec((4, TN, 128), lambda i: (0, i, 0)),
        out_shape=jax.ShapeDtypeStruct((4, N, 128), jnp.float32),
    )(deg2, agg1, xq, W1, b1, W2)


def _tc_layer2(deg2, agg2, y2, b2, W3):
    def body(deg_ref, a_ref, y_ref, b2_ref, w3_ref, o_ref):
        dis = _dis(deg_ref)
        s = jnp.concatenate([a_ref[q] + y_ref[q] for q in range(4)], axis=1)
        h = jnp.maximum(s * dis + b2_ref[...], 0.0)
        y = jnp.dot(h, w3_ref[...], preferred_element_type=jnp.float32) * dis
        for q in range(4):
            o_ref[q] = y[:, 128 * q:128 * (q + 1)]

    return pl.pallas_call(
        body,
        grid=(GRID,),
        in_specs=[
            pl.BlockSpec((2, TN, 128), lambda i: (0, i, 0)),
            pl.BlockSpec((4, TN, 128), lambda i: (0, i, 0)),
            pl.BlockSpec((4, TN, 128), lambda i: (0, i, 0)),
            pl.BlockSpec((1, H), lambda i: (0, 0)),
            pl.BlockSpec((H, H), lambda i: (0, 0)),
        ],
        out_specs=pl.BlockSpec((4, TN, 128), lambda i: (0, i, 0)),
        out_shape=jax.ShapeDtypeStruct((4, N, 128), jnp.float32),
    )(deg2, agg2, y2, b2, W3)


def _tc_final(deg2, agg3, y3, b3):
    def body(deg_ref, a_ref, y_ref, b3_ref, o_ref):
        dis = _dis(deg_ref)
        s = jnp.concatenate([a_ref[q] + y_ref[q] for q in range(4)], axis=1)
        o_ref[...] = s * dis + b3_ref[...]

    return pl.pallas_call(
        body,
        grid=(GRID,),
        in_specs=[
            pl.BlockSpec((2, TN, 128), lambda i: (0, i, 0)),
            pl.BlockSpec((4, TN, 128), lambda i: (0, i, 0)),
            pl.BlockSpec((4, TN, 128), lambda i: (0, i, 0)),
            pl.BlockSpec((1, H), lambda i: (0, 0)),
        ],
        out_specs=pl.BlockSpec((TN, H), lambda i: (i, 0)),
        out_shape=jax.ShapeDtypeStruct((N, H), jnp.float32),
    )(deg2, agg3, y3, b3)


def kernel(x, edge_index, batch, W1, b1, W2, b2, W3, b3):
    src = edge_index[0]
    dst = edge_index[1]
    z_nk = jnp.zeros((N, 128), jnp.float32)
    ones_kk = jnp.ones((K, 128), jnp.float32)

    deg2 = _sc_degree(dst, ones_kk)

    xq = _tc_pre(deg2, x)
    agg1 = _sc_agg(xq, src, dst, z_nk, 2)
    y2 = _tc_layer1(deg2, agg1, xq, W1.reshape(D_IN, H), b1.reshape(1, H), W2)
    agg2 = _sc_agg(y2, src, dst, z_nk, 4)
    y3 = _tc_layer2(deg2, agg2, y2, b2.reshape(1, H), W3)
    agg3 = _sc_agg(y3, src, dst, z_nk, 4)
    out = _tc_final(deg2, agg3, y3, b3.reshape(1, H))
    return (out, out)


# speedup vs baseline: 13.6539x; 1.0087x over previous
import functools

import jax
import jax.numpy as jnp
from jax import lax
from jax.experimental import pallas as pl
from jax.experimental.pallas import tpu as pltpu
from jax.experimental.pallas import tpu_sc as plsc

N = 10000
E = 160000
D_IN = 256
H = 512
K = 128
NCHUNK = E // K
TN = 400
GRID = N // TN

_mesh = plsc.VectorSubcoreMesh(core_axis_name="c", subcore_axis_name="s")


def _zero_rows(z_h, acc, sid):
    @pl.when(sid < 15)
    def _():
        pltpu.sync_copy(z_h.at[pl.ds(sid * 640, 640)],
                        acc.at[pl.ds(sid * 640, 640)])

    @pl.when(sid == 15)
    def _():
        pltpu.sync_copy(z_h.at[pl.ds(9600, 400)], acc.at[pl.ds(9600, 400)])


def _sc_degree(dst, ones_kk):
    DEPTH = 3

    @functools.partial(
        pl.kernel,
        out_type=jax.ShapeDtypeStruct((2, N, 128), jnp.float32),
        mesh=_mesh,
        scratch_types=[
            pltpu.VMEM((DEPTH, K), jnp.int32),
            pltpu.VMEM((K, 128), jnp.float32),
            pltpu.VMEM((K, 128), jnp.float32),
            pltpu.VMEM_SHARED((N, 128), jnp.float32),
            pltpu.SemaphoreType.DMA((DEPTH,)),
            pltpu.SemaphoreType.DMA((DEPTH,)),
            pltpu.SemaphoreType.DMA,
        ],
    )
    def k(dst_h, ones_h, out_h, dst_v, ones_v, zero_v, acc,
          sem_j, sem_s, sem_z):
        c = lax.axis_index("c")
        sid = lax.axis_index("s")
        pltpu.sync_copy(ones_h, ones_v)

        @pl.loop(0, K)
        def _(i):
            @pl.loop(0, 128, step=16)
            def _(j):
                zero_v[i, pl.ds(j, 16)] = jnp.zeros((16,), jnp.float32)

        @pl.when(sid < 15)
        def _():
            zc = [pltpu.async_copy(
                zero_v, acc.at[pl.ds(sid * 640 + t * 128, 128)], sem_z)
                for t in range(5)]
            for z in zc:
                z.wait()

        @pl.when(sid == 15)
        def _():
            zc = [pltpu.async_copy(
                zero_v, acc.at[pl.ds(9600 + t * 128, 128)], sem_z)
                for t in range(3)]
            zc.append(pltpu.async_copy(
                zero_v.at[pl.ds(0, 16)], acc.at[pl.ds(9984, 16)], sem_z))
            for z in zc:
                z.wait()

        plsc.subcore_barrier()

        base = c * (NCHUNK // 2)

        def batch(r, first):
            jcps = []
            for i in range(DEPTH):
                if not first:
                    pltpu.make_async_copy(
                        ones_v, acc.at[dst_v.at[i]], sem_s.at[i]).wait()
                cid = base + (r + i) * 16 + sid
                jcps.append(pltpu.async_copy(
                    dst_h.at[pl.ds(cid * K, K)], dst_v.at[i], sem_j.at[i]))
            for i in range(DEPTH):
                jcps[i].wait()
                pltpu.async_copy(
                    ones_v, acc.at[dst_v.at[i]], sem_s.at[i], add=True)

        batch(0, first=True)

        @pl.loop(DEPTH, 39, step=DEPTH)
        def _(r):
            batch(r, first=False)

        for i in range(DEPTH):
            pltpu.make_async_copy(
                ones_v, acc.at[dst_v.at[i]], sem_s.at[i]).wait()

        @pl.when(sid == 0)
        def _():
            cid = base + 624
            pltpu.sync_copy(dst_h.at[pl.ds(cid * K, K)], dst_v.at[0])
            pltpu.sync_copy(ones_v, acc.at[dst_v.at[0]], add=True)

        plsc.subcore_barrier()

        @pl.when(sid < 15)
        def _():
            pltpu.sync_copy(acc.at[pl.ds(sid * 640, 640)],
                            out_h.at[c].at[pl.ds(sid * 640, 640)])

        @pl.when(sid == 15)
        def _():
            pltpu.sync_copy(acc.at[pl.ds(9600, 400)],
                            out_h.at[c].at[pl.ds(9600, 400)])

    return k(dst, ones_kk)


def _sc_agg(table, src, dst, z_nk, C):
    DEPTH = 3

    @functools.partial(
        pl.kernel,
        out_type=jax.ShapeDtypeStruct((C, N, 128), jnp.float32),
        mesh=_mesh,
        scratch_types=[
            pltpu.VMEM((DEPTH, K), jnp.int32),
            pltpu.VMEM((DEPTH, K), jnp.int32),
            pltpu.VMEM((DEPTH, K, 128), jnp.float32),
            pltpu.VMEM_SHARED((N, 128), jnp.float32),
            pltpu.SemaphoreType.DMA((DEPTH,)),
            pltpu.SemaphoreType.DMA((DEPTH,)),
            pltpu.SemaphoreType.DMA((DEPTH,)),
            pltpu.SemaphoreType.DMA((DEPTH,)),
            pltpu.SemaphoreType.DMA,
        ],
    )
    def k(table_h, src_h, dst_h, z_h, out_h, src_v, dst_v, msgs_v, acc,
          sem_i, sem_j, sem_g, sem_s, sem_z):
        c = lax.axis_index("c")
        sid = lax.axis_index("s")
        NQ = C // 2
        qs = [c * NQ + qq for qq in range(NQ)]

        def zero_issue():
            @pl.when(sid < 15)
            def _():
                pltpu.async_copy(z_h.at[pl.ds(sid * 640, 640)],
                                 acc.at[pl.ds(sid * 640, 640)], sem_z)

            @pl.when(sid == 15)
            def _():
                pltpu.async_copy(z_h.at[pl.ds(9600, 400)],
                                 acc.at[pl.ds(9600, 400)], sem_z)

        def zero_wait():
            @pl.when(sid < 15)
            def _():
                pltpu.make_async_copy(z_h.at[pl.ds(sid * 640, 640)],
                                      acc.at[pl.ds(sid * 640, 640)],
                                      sem_z).wait()

            @pl.when(sid == 15)
            def _():
                pltpu.make_async_copy(z_h.at[pl.ds(9600, 400)],
                                      acc.at[pl.ds(9600, 400)], sem_z).wait()

        def issue_idx(r, ring):
            icps, jcps = [], []
            for i in range(DEPTH):
                if ring:
                    pltpu.make_async_copy(
                        msgs_v.at[i], acc.at[dst_v.at[i]],
                        sem_s.at[i]).wait()
                cid = (r + i) * 16 + sid
                icps.append(pltpu.async_copy(
                    src_h.at[pl.ds(cid * K, K)], src_v.at[i], sem_i.at[i]))
                jcps.append(pltpu.async_copy(
                    dst_h.at[pl.ds(cid * K, K)], dst_v.at[i], sem_j.at[i]))
            return icps, jcps

        def issue_gathers(icps, q):
            gcps = []
            for i in range(DEPTH):
                icps[i].wait()
                gcps.append(pltpu.async_copy(
                    table_h.at[q].at[src_v.at[i]], msgs_v.at[i],
                    sem_g.at[i]))
            return gcps

        def issue_scatters(gcps, jcps):
            for i in range(DEPTH):
                gcps[i].wait()
                jcps[i].wait()
                pltpu.async_copy(
                    msgs_v.at[i], acc.at[dst_v.at[i]], sem_s.at[i],
                    add=True)

        def drain(q):
            @pl.when(sid < 15)
            def _():
                pltpu.sync_copy(acc.at[pl.ds(sid * 640, 640)],
                                out_h.at[q].at[pl.ds(sid * 640, 640)])

            @pl.when(sid == 15)
            def _():
                pltpu.sync_copy(acc.at[pl.ds(9600, 400)],
                                out_h.at[q].at[pl.ds(9600, 400)])

        zero_issue()
        icps, jcps = issue_idx(0, ring=False)
        gcps = issue_gathers(icps, qs[0])
        zero_wait()
        plsc.subcore_barrier()

        for qq in range(NQ):
            q = qs[qq]
            issue_scatters(gcps, jcps)

            @pl.loop(DEPTH, NCHUNK // 16, step=DEPTH)
            def _(r, q=q):
                ic, jc = issue_idx(r, ring=True)
                issue_scatters(issue_gathers(ic, q), jc)

            for i in range(DEPTH):
                pltpu.make_async_copy(
                    msgs_v.at[i], acc.at[dst_v.at[i]], sem_s.at[i]).wait()

            @pl.when(sid < NCHUNK - (NCHUNK // 16) * 16)
            def _(q=q):
                cid = (NCHUNK // 16) * 16 + sid
                pltpu.sync_copy(src_h.at[pl.ds(cid * K, K)], src_v.at[0])
                pltpu.sync_copy(dst_h.at[pl.ds(cid * K, K)], dst_v.at[0])
                pltpu.sync_copy(table_h.at[q].at[src_v.at[0]], msgs_v.at[0])
                pltpu.sync_copy(msgs_v.at[0], acc.at[dst_v.at[0]], add=True)

            plsc.subcore_barrier()

            if qq + 1 < NQ:
                icps, jcps = issue_idx(0, ring=False)
                gcps = issue_gathers(icps, qs[qq + 1])
                drain(q)
                zero_issue()
                zero_wait()
                plsc.subcore_barrier()
            else:
                drain(q)

    return k(table, src, dst, z_nk)


def _dis(deg_ref):
    deg = deg_ref[0, :, 0:1] + deg_ref[1, :, 0:1] + 1.0
    return lax.rsqrt(deg)


def _tc_pre(deg2, x):
    def body(deg_ref, x_ref, xq_ref):
        xs = x_ref[...] * _dis(deg_ref)
        xq_ref[0] = xs[:, :128]
        xq_ref[1] = xs[:, 128:]

    return pl.pallas_call(
        body,
        grid=(GRID,),
        in_specs=[
            pl.BlockSpec((2, TN, 128), lambda i: (0, i, 0)),
            pl.BlockSpec((TN, D_IN), lambda i: (i, 0)),
        ],
        out_specs=pl.BlockSpec((2, TN, 128), lambda i: (0, i, 0)),
        out_shape=jax.ShapeDtypeStruct((2, N, 128), jnp.float32),
    )(deg2, x)


def _tc_layer1(deg2, agg1, xq, W1, b1, W2):
    def body(deg_ref, a_ref, x_ref, w1_ref, b1_ref, w2_ref, y_ref):
        dis = _dis(deg_ref)
        z = jnp.concatenate(
            [a_ref[0] + x_ref[0], a_ref[1] + x_ref[1]], axis=1) * dis
        h = jnp.maximum(
            jnp.dot(z, w1_ref[...], preferred_element_type=jnp.float32)
            + b1_ref[...], 0.0)
        y = jnp.dot(h, w2_ref[...], preferred_element_type=jnp.float32) * dis
        for q in range(4):
            y_ref[q] = y[:, 128 * q:128 * (q + 1)]

    return pl.pallas_call(
        body,
        grid=(GRID,),
        in_specs=[
            pl.BlockSpec((2, TN, 128), lambda i: (0, i, 0)),
            pl.BlockSpec((2, TN, 128), lambda i: (0, i, 0)),
            pl.BlockSpec((2, TN, 128), lambda i: (0, i, 0)),
            pl.BlockSpec((D_IN, H), lambda i: (0, 0)),
            pl.BlockSpec((1, H), lambda i: (0, 0)),
            pl.BlockSpec((H, H), lambda i: (0, 0)),
        ],
        out_specs=pl.BlockSpec((4, TN, 128), lambda i: (0, i, 0)),
        out_shape=jax.ShapeDtypeStruct((4, N, 128), jnp.float32),
    )(deg2, agg1, xq, W1, b1, W2)


def _tc_layer2(deg2, agg2, y2, b2, W3):
    def body(deg_ref, a_ref, y_ref, b2_ref, w3_ref, o_ref):
        dis = _dis(deg_ref)
        s = jnp.concatenate([a_ref[q] + y_ref[q] for q in range(4)], axis=1)
        h = jnp.maximum(s * dis + b2_ref[...], 0.0)
        y = jnp.dot(h, w3_ref[...], preferred_element_type=jnp.float32) * dis
        for q in range(4):
            o_ref[q] = y[:, 128 * q:128 * (q + 1)]

    return pl.pallas_call(
        body,
        grid=(GRID,),
        in_specs=[
            pl.BlockSpec((2, TN, 128), lambda i: (0, i, 0)),
            pl.BlockSpec((4, TN, 128), lambda i: (0, i, 0)),
            pl.BlockSpec((4, TN, 128), lambda i: (0, i, 0)),
            pl.BlockSpec((1, H), lambda i: (0, 0)),
            pl.BlockSpec((H, H), lambda i: (0, 0)),
        ],
        out_specs=pl.BlockSpec((4, TN, 128), lambda i: (0, i, 0)),
        out_shape=jax.ShapeDtypeStruct((4, N, 128), jnp.float32),
    )(deg2, agg2, y2, b2, W3)


def _tc_final(deg2, agg3, y3, b3):
    def body(deg_ref, a_ref, y_ref, b3_ref, o_ref):
        dis = _dis(deg_ref)
        s = jnp.concatenate([a_ref[q] + y_ref[q] for q in range(4)], axis=1)
        o_ref[...] = s * dis + b3_ref[...]

    return pl.pallas_call(
        body,
        grid=(GRID,),
        in_specs=[
            pl.BlockSpec((2, TN, 128), lambda i: (0, i, 0)),
            pl.BlockSpec((4, TN, 128), lambda i: (0, i, 0)),
            pl.BlockSpec((4, TN, 128), lambda i: (0, i, 0)),
            pl.BlockSpec((1, H), lambda i: (0, 0)),
        ],
        out_specs=pl.BlockSpec((TN, H), lambda i: (i, 0)),
        out_shape=jax.ShapeDtypeStruct((N, H), jnp.float32),
    )(deg2, agg3, y3, b3)


def kernel(x, edge_index, batch, W1, b1, W2, b2, W3, b3):
    src = edge_index[0]
    dst = edge_index[1]
    z_nk = jnp.zeros((N, 128), jnp.float32)
    ones_kk = jnp.ones((K, 128), jnp.float32)

    deg2 = _sc_degree(dst, ones_kk)

    xq = _tc_pre(deg2, x)
    agg1 = _sc_agg(xq, src, dst, z_nk, 2)
    y2 = _tc_layer1(deg2, agg1, xq, W1.reshape(D_IN, H), b1.reshape(1, H), W2)
    agg2 = _sc_agg(y2, src, dst, z_nk, 4)
    y3 = _tc_layer2(deg2, agg2, y2, b2.reshape(1, H), W3)
    agg3 = _sc_agg(y3, src, dst, z_nk, 4)
    out = _tc_final(deg2, agg3, y3, b3.reshape(1, H))
    return (out, out)
